# baseline (speedup 1.0000x reference)
import jax
import jax.numpy as jnp
from jax.experimental import pallas as pl


_BC = 256
_CH = 8


def _getmax_block(w_ref, o_ref):
    n = w_ref.shape[0]
    sub = jax.lax.broadcasted_iota(jnp.int32, (_CH, _BC), 0)

    def step(i, carry):
        best_a, best_r = carry
        xc = w_ref[pl.ds(i * _CH, _CH), :]
        ac = jnp.abs(xc)
        upd = ac > best_a
        best_a = jnp.where(upd, ac, best_a)
        best_r = jnp.where(upd, sub + i * _CH, best_r)
        return best_a, best_r

    init = (jnp.full((_CH, _BC), -1.0, jnp.float32),
            jnp.zeros((_CH, _BC), jnp.int32))
    best_a, best_r = jax.lax.fori_loop(0, n // _CH, step, init)

    m = jnp.max(best_a, axis=0, keepdims=True)
    first = jnp.min(jnp.where(best_a == m, best_r, n), axis=0, keepdims=True)

    rows = jax.lax.broadcasted_iota(jnp.int32, (n, _BC), 0)
    o_ref[:, :] = jnp.where(rows == first, w_ref[:, :], 0.0)


def kernel(w):
    n, mcols = w.shape
    grid = (mcols // _BC,)
    return pl.pallas_call(
        _getmax_block,
        grid=grid,
        in_specs=[pl.BlockSpec((n, _BC), lambda j: (0, j))],
        out_specs=pl.BlockSpec((n, _BC), lambda j: (0, j)),
        out_shape=jax.ShapeDtypeStruct((n, mcols), w.dtype),
    )(w)


# speedup vs baseline: 1.0234x; 1.0234x over previous
import jax
import jax.numpy as jnp
from jax.experimental import pallas as pl
from jax.experimental.pallas import tpu as pltpu


_BC = 128


def _getmax_block(w_ref, o_ref):
    x = w_ref[:, :]
    a = jnp.abs(x)
    m = jnp.max(a, axis=0, keepdims=True)
    rows = jax.lax.broadcasted_iota(jnp.int32, x.shape, 0)
    masked_rows = jnp.where(a == m, rows, x.shape[0])
    first = jnp.min(masked_rows, axis=0, keepdims=True)
    o_ref[:, :] = jnp.where(rows == first, x, 0.0)


def kernel(w):
    n, mcols = w.shape
    grid = (mcols // _BC,)
    return pl.pallas_call(
        _getmax_block,
        grid=grid,
        in_specs=[pl.BlockSpec((n, _BC), lambda j: (0, j))],
        out_specs=pl.BlockSpec((n, _BC), lambda j: (0, j)),
        out_shape=jax.ShapeDtypeStruct((n, mcols), w.dtype),
        compiler_params=pltpu.CompilerParams(
            vmem_limit_bytes=100 * 1024 * 1024,
        ),
    )(w)


# speedup vs baseline: 1.2727x; 1.2436x over previous
import jax
import jax.numpy as jnp
from jax.experimental import pallas as pl


_BC = 256
_CH = 64


def _getmax_block(w_ref, o_ref):
    n = w_ref.shape[0]
    nch = n // _CH

    m = None
    for c in range(nch):
        ac = jnp.abs(w_ref[pl.ds(c * _CH, _CH), :])
        lm = jnp.max(ac, axis=0, keepdims=True)
        m = lm if m is None else jnp.maximum(m, lm)

    sub = jax.lax.broadcasted_iota(jnp.int32, (_CH, _BC), 0)
    first = None
    for c in range(nch):
        ac = jnp.abs(w_ref[pl.ds(c * _CH, _CH), :])
        lf = jnp.min(jnp.where(ac == m, sub + c * _CH, n),
                     axis=0, keepdims=True)
        first = lf if first is None else jnp.minimum(first, lf)

    for c in range(nch):
        xc = w_ref[pl.ds(c * _CH, _CH), :]
        o_ref[pl.ds(c * _CH, _CH), :] = jnp.where(
            sub + c * _CH == first, xc, 0.0)


def kernel(w):
    n, mcols = w.shape
    grid = (mcols // _BC,)
    return pl.pallas_call(
        _getmax_block,
        grid=grid,
        in_specs=[pl.BlockSpec((n, _BC), lambda j: (0, j))],
        out_specs=pl.BlockSpec((n, _BC), lambda j: (0, j)),
        out_shape=jax.ShapeDtypeStruct((n, mcols), w.dtype),
    )(w)


# speedup vs baseline: 1.2782x; 1.0043x over previous
import jax
import jax.numpy as jnp
from jax.experimental import pallas as pl


_BC = 256
_CH = 64


def _getmax_block(w_ref, o_ref):
    n = w_ref.shape[0]
    nch = n // _CH

    sub = jax.lax.broadcasted_iota(jnp.int32, (_CH, _BC), 0)
    m = jnp.full((1, _BC), -1.0, jnp.float32)
    first = jnp.full((1, _BC), n, jnp.int32)
    for c in range(nch):
        ac = jnp.abs(w_ref[pl.ds(c * _CH, _CH), :])
        lm = jnp.max(ac, axis=0, keepdims=True)
        lf = jnp.min(jnp.where(ac == lm, sub + c * _CH, n),
                     axis=0, keepdims=True)
        upd = lm > m
        m = jnp.where(upd, lm, m)
        first = jnp.where(upd, lf, first)

    for c in range(nch):
        xc = w_ref[pl.ds(c * _CH, _CH), :]
        o_ref[pl.ds(c * _CH, _CH), :] = jnp.where(
            sub + c * _CH == first, xc, 0.0)


def kernel(w):
    n, mcols = w.shape
    grid = (mcols // _BC,)
    return pl.pallas_call(
        _getmax_block,
        grid=grid,
        in_specs=[pl.BlockSpec((n, _BC), lambda j: (0, j))],
        out_specs=pl.BlockSpec((n, _BC), lambda j: (0, j)),
        out_shape=jax.ShapeDtypeStruct((n, mcols), w.dtype),
    )(w)
